# baseline (speedup 1.0000x reference)
import functools

import jax
import jax.numpy as jnp
from jax import lax
from jax.experimental import pallas as pl
from jax.experimental.pallas import tpu as pltpu
from jax.experimental.pallas import tpu_sc as plsc

_INFO = plsc.get_sparse_core_info()
NW = _INFO.num_cores * _INFO.num_subcores
CHUNK = 128


@functools.lru_cache(maxsize=None)
def _build(n_rows: int, dim: int, vocab: int):
    per_w = n_rows // (NW * CHUNK)
    assert per_w * NW * CHUNK == n_rows and per_w >= 3

    mesh = plsc.VectorSubcoreMesh(core_axis_name="c", subcore_axis_name="s")

    @functools.partial(
        pl.kernel,
        mesh=mesh,
        out_type=jax.ShapeDtypeStruct((n_rows, dim), jnp.float32),
        scratch_types=[
            pltpu.VMEM((3, CHUNK), jnp.int32),
            pltpu.VMEM((3, CHUNK, 2 * dim), jnp.float32),
            pltpu.VMEM((2, CHUNK, dim), jnp.float32),
            pltpu.SemaphoreType.DMA,
            pltpu.SemaphoreType.DMA,
            pltpu.SemaphoreType.DMA,
        ],
    )
    def gather_kernel(xr, emb_p, out, idxb, gbuf, obuf, isem, gsem, osem):
        w = lax.axis_index("s") * _INFO.num_cores + lax.axis_index("c")

        def start_idx(t, buf):
            pltpu.async_copy(xr.at[w, t], idxb.at[buf], isem)

        def wait_idx(t, buf):
            pltpu.make_async_copy(xr.at[w, t], idxb.at[buf], isem).wait()

        def start_gather(buf):
            pltpu.async_copy(emb_p.at[idxb.at[buf]], gbuf.at[buf], gsem)

        def wait_gather(buf):
            pltpu.make_async_copy(
                emb_p.at[idxb.at[buf]], gbuf.at[buf], gsem
            ).wait()

        def compact(gb, ob):
            segs = dim // 16

            def rbody(r, carry, gb=gb, ob=ob):
                vals = [
                    gbuf[gb, r, pl.ds(s * 16, 16)] for s in range(segs)
                ]
                for s in range(segs):
                    obuf[ob, r, pl.ds(s * 16, 16)] = vals[s]
                return carry

            lax.fori_loop(0, CHUNK, rbody, 0)

        def start_owrite(t, buf):
            row = pl.multiple_of((w * per_w + t) * CHUNK, CHUNK)
            pltpu.async_copy(
                obuf.at[buf], out.at[pl.ds(row, CHUNK), :], osem
            )

        def wait_owrite():
            pltpu.make_async_copy(
                obuf.at[0], out.at[pl.ds(0, CHUNK), :], osem
            ).wait()

        start_idx(0, 0)
        start_idx(1, 1)
        wait_idx(0, 0)
        start_gather(0)

        def body(t, carry):
            @pl.when(t + 2 < per_w)
            def _():
                start_idx(t + 2, lax.rem(t + 2, 3))

            @pl.when(t >= 2)
            def _():
                wait_owrite()

            @pl.when(t + 1 < per_w)
            def _():
                wait_idx(t + 1, lax.rem(t + 1, 3))
                start_gather(lax.rem(t + 1, 3))

            wait_gather(lax.rem(t, 3))
            compact(lax.rem(t, 3), lax.rem(t, 2))
            start_owrite(t, lax.rem(t, 2))
            return carry

        lax.fori_loop(0, per_w, body, 0)
        wait_owrite()
        wait_owrite()

    return gather_kernel


def kernel(x, emb):
    bsz, hist = x.shape
    vocab, dim = emb.shape
    n_rows = bsz * hist
    emb_p = jnp.zeros((vocab, 128), emb.dtype).at[:, :dim].set(emb)
    per_w = n_rows // (NW * CHUNK)
    xr = x.astype(jnp.int32).reshape(NW, per_w, CHUNK)
    out = _build(n_rows, dim, vocab)(xr, emb_p)
    return out.reshape(bsz, hist, dim)


# speedup vs baseline: 1.3225x; 1.3225x over previous
import functools

import jax
import jax.numpy as jnp
from jax import lax
from jax.experimental import pallas as pl
from jax.experimental.pallas import tpu as pltpu
from jax.experimental.pallas import tpu_sc as plsc

_INFO = plsc.get_sparse_core_info()
NW = _INFO.num_cores * _INFO.num_subcores
CHUNK = 128


@functools.lru_cache(maxsize=None)
def _build(n_rows: int, dim: int, vocab: int):
    per_w = n_rows // (NW * CHUNK)
    assert per_w * NW * CHUNK == n_rows and per_w >= 3

    mesh = plsc.VectorSubcoreMesh(core_axis_name="c", subcore_axis_name="s")

    @functools.partial(
        pl.kernel,
        mesh=mesh,
        out_type=jax.ShapeDtypeStruct((n_rows, dim), jnp.float32),
        scratch_types=[
            pltpu.VMEM((4, CHUNK), jnp.int32),
            pltpu.VMEM((4, CHUNK, 2 * dim), jnp.float32),
            pltpu.VMEM((2, CHUNK, dim), jnp.float32),
            pltpu.SemaphoreType.DMA,
            pltpu.SemaphoreType.DMA,
            pltpu.SemaphoreType.DMA,
        ],
    )
    def gather_kernel(xr, emb_p, out, idxb, gbuf, obuf, isem, gsem, osem):
        w = lax.axis_index("s") * _INFO.num_cores + lax.axis_index("c")

        def start_idx(t, buf):
            pltpu.async_copy(xr.at[w, t], idxb.at[buf], isem)

        def wait_idx(t, buf):
            pltpu.make_async_copy(xr.at[w, t], idxb.at[buf], isem).wait()

        def start_gather(buf):
            pltpu.async_copy(emb_p.at[idxb.at[buf]], gbuf.at[buf], gsem)

        def wait_gather(buf):
            pltpu.make_async_copy(
                emb_p.at[idxb.at[buf]], gbuf.at[buf], gsem
            ).wait()

        def compact(gb, ob):
            segs = dim // 16

            def rbody(r, carry, gb=gb, ob=ob):
                vals = [
                    gbuf[gb, r, pl.ds(s * 16, 16)] for s in range(segs)
                ]
                for s in range(segs):
                    obuf[ob, r, pl.ds(s * 16, 16)] = vals[s]
                return carry

            lax.fori_loop(0, CHUNK, rbody, 0)

        def start_owrite(t, buf):
            row = pl.multiple_of((w * per_w + t) * CHUNK, CHUNK)
            pltpu.async_copy(
                obuf.at[buf], out.at[pl.ds(row, CHUNK), :], osem
            )

        def wait_owrite():
            pltpu.make_async_copy(
                obuf.at[0], out.at[pl.ds(0, CHUNK), :], osem
            ).wait()

        start_idx(0, 0)
        start_idx(1, 1)
        start_idx(2, 2)
        wait_idx(0, 0)
        start_gather(0)
        wait_idx(1, 1)
        start_gather(1)

        def body(t, carry):
            @pl.when(t + 3 < per_w)
            def _():
                start_idx(t + 3, lax.rem(t + 3, 4))

            @pl.when(t >= 2)
            def _():
                wait_owrite()

            @pl.when(t + 2 < per_w)
            def _():
                wait_idx(t + 2, lax.rem(t + 2, 4))
                start_gather(lax.rem(t + 2, 4))

            wait_gather(lax.rem(t, 4))
            compact(lax.rem(t, 4), lax.rem(t, 2))
            start_owrite(t, lax.rem(t, 2))
            return carry

        lax.fori_loop(0, per_w, body, 0)
        wait_owrite()
        wait_owrite()

    return gather_kernel


def kernel(x, emb):
    bsz, hist = x.shape
    vocab, dim = emb.shape
    n_rows = bsz * hist
    emb_p = jnp.pad(emb, ((0, 0), (0, 128 - dim)))
    per_w = n_rows // (NW * CHUNK)
    xr = x.astype(jnp.int32).reshape(NW, per_w, CHUNK)
    out = _build(n_rows, dim, vocab)(xr, emb_p)
    return out.reshape(bsz, hist, dim)
